# baseline (speedup 1.0000x reference)
import functools

import jax
import jax.numpy as jnp
from jax import lax
from jax.experimental import pallas as pl
from jax.experimental.pallas import tpu as pltpu
from jax.experimental.pallas import tpu_sc as plsc

N = 10000
K = 32
C = 128
NP = 10240
NW = 32
PPW = NP // NW
CPC = 4
NCH = PPW // CPC
NBUF = 4
TN = 2048
KS3 = 125


def _sc_gather_sum(feaT, idx):
    mesh = plsc.VectorSubcoreMesh(
        core_axis_name="c", subcore_axis_name="s", num_cores=2, num_subcores=16
    )

    @functools.partial(
        pl.kernel,
        out_type=jax.ShapeDtypeStruct((NP, C), jnp.float32),
        mesh=mesh,
        scratch_types=[
            pltpu.VMEM((NCH, 128), jnp.int32),
            pltpu.VMEM((128, C), jnp.float32),
            pltpu.VMEM((128, C), jnp.float32),
            pltpu.VMEM((128, C), jnp.float32),
            pltpu.VMEM((128, C), jnp.float32),
            pltpu.VMEM((NBUF, 128), jnp.int32),
            pltpu.VMEM_SHARED((16 * PPW, C), jnp.float32),
            pltpu.SemaphoreType.DMA,
            pltpu.SemaphoreType.DMA,
            pltpu.SemaphoreType.DMA,
            pltpu.SemaphoreType.DMA,
            pltpu.SemaphoreType.DMA,
            pltpu.SemaphoreType.DMA,
            pltpu.SemaphoreType.DMA,
            pltpu.SemaphoreType.DMA,
        ],
    )
    def sc_kernel(
        feaT_hbm, idx_hbm, out_hbm,
        idx_v, rows0, rows1, rows2, rows3, ids_v, acc_sh,
        gsem0, gsem1, gsem2, gsem3, ssem0, ssem1, ssem2, ssem3,
    ):
        sid = lax.axis_index("s")
        wid = sid * 2 + lax.axis_index("c")
        sbase = sid * PPW
        pltpu.sync_copy(idx_hbm.at[wid], idx_v)

        for r in range(128):
            for v in range(C // 16):
                rows0[r, pl.ds(v * 16, 16)] = jnp.zeros((16,), jnp.float32)
        pltpu.sync_copy(rows0, acc_sh.at[pl.ds(sbase, 128)])
        pltpu.sync_copy(rows0, acc_sh.at[pl.ds(sbase + 128, 128)])
        pltpu.sync_copy(rows0.at[pl.ds(0, 64)], acc_sh.at[pl.ds(sbase + 256, 64)])

        rows = (rows0, rows1, rows2, rows3)
        gsems = (gsem0, gsem1, gsem2, gsem3)
        ssems = (ssem0, ssem1, ssem2, ssem3)

        for b in range(NBUF):
            pltpu.async_copy(feaT_hbm.at[idx_v.at[b]], rows[b], gsems[b])

        def group(i, carry):
            c0 = i * NBUF
            for b in range(NBUF):
                c = c0 + b
                rb = rows[b]
                pltpu.make_async_copy(feaT_hbm.at[idx_v.at[c]], rb, gsems[b]).wait()
                for v in range(128 // 16):
                    ids_v[b, pl.ds(v * 16, 16)] = jnp.full(
                        (16,), sbase + c * CPC + v // 2, dtype=jnp.int32
                    )
                pltpu.async_copy(rb, acc_sh.at[ids_v.at[b]], ssems[b], add=True)
            for b in range(NBUF):
                c = c0 + b
                rb = rows[b]
                pltpu.make_async_copy(rb, acc_sh.at[ids_v.at[b]], ssems[b]).wait()

                @pl.when(c + NBUF < NCH)
                def _():
                    pltpu.async_copy(feaT_hbm.at[idx_v.at[c + NBUF]], rb, gsems[b])

            return carry

        lax.fori_loop(0, NCH // NBUF, group, 0)
        pltpu.sync_copy(
            acc_sh.at[pl.ds(sbase, PPW)], out_hbm.at[pl.ds(wid * PPW, PPW)]
        )

    return sc_kernel(feaT, idx)


def _tc_body(s_ref, fea_ref, rel_ref, smp_ref, w1f_ref, w1x_ref, b1_ref, dw_ref, out_ref):
    w1f = w1f_ref[:]
    z = lax.dot_general(
        w1f, s_ref[:], (((1,), (1,)), ((), ())), preferred_element_type=jnp.float32
    )
    z = z - 30.0 * jnp.dot(w1f, fea_ref[:], preferred_element_type=jnp.float32)
    rel_s = jnp.sum(rel_ref[:], axis=1)
    z = z + jnp.dot(w1x_ref[:], rel_s, preferred_element_type=jnp.float32)
    z = z + 32.0 * b1_ref[:]

    smp = jnp.clip(smp_ref[:], -0.99999, 0.99999) * (5.0 / 2.0)
    coord = smp.astype(jnp.int32) + 2
    pos = coord[2:3, :] * 25 + coord[1:2, :] * 5 + coord[0:1, :]
    onehot = (
        lax.broadcasted_iota(jnp.int32, (KS3, TN), 0) == pos
    ).astype(jnp.float32)
    kern = jnp.dot(dw_ref[:], onehot, preferred_element_type=jnp.float32)
    out_ref[:] = kern * z


def _tc_combine(S, fea2d, rel3, smpT, W1f, W1x, b1c, dw):
    return pl.pallas_call(
        _tc_body,
        grid=((N + TN - 1) // TN,),
        in_specs=[
            pl.BlockSpec((TN, C), lambda i: (i, 0)),
            pl.BlockSpec((C, TN), lambda i: (0, i)),
            pl.BlockSpec((3, K, TN), lambda i: (0, 0, i)),
            pl.BlockSpec((3, TN), lambda i: (0, i)),
            pl.BlockSpec((C, C), lambda i: (0, 0)),
            pl.BlockSpec((C, 3), lambda i: (0, 0)),
            pl.BlockSpec((C, 1), lambda i: (0, 0)),
            pl.BlockSpec((C, KS3), lambda i: (0, 0)),
        ],
        out_specs=pl.BlockSpec((C, TN), lambda i: (0, i)),
        out_shape=jax.ShapeDtypeStruct((C, N), jnp.float32),
    )(S, fea2d, rel3, smpT, W1f, W1x, b1c, dw)


def kernel(sample_xyz, rel_xyz, fea, knn_idx, W1, b1, conv_dw):
    fea2d = fea[0]
    feaT = jnp.pad(fea2d.T, ((0, NP - N), (0, 0)))
    idx = knn_idx[0, :, 1:]
    idx = jnp.concatenate(
        [idx, jnp.full((N, 1), N, dtype=jnp.int32)], axis=1
    )
    idx = jnp.pad(idx, ((0, NP - N), (0, 0)), constant_values=N)
    idx = idx.reshape(NW, NCH, 128)

    S = _sc_gather_sum(feaT, idx)

    out2d = _tc_combine(
        S,
        fea2d,
        rel_xyz[0],
        sample_xyz[0].T,
        W1[:, :C],
        W1[:, C:],
        b1.reshape(C, 1),
        conv_dw[0].reshape(C, KS3),
    )
    return out2d[None]


# speedup vs baseline: 7.7500x; 7.7500x over previous
import functools

import jax
import jax.numpy as jnp
from jax import lax
from jax.experimental import pallas as pl
from jax.experimental.pallas import tpu as pltpu
from jax.experimental.pallas import tpu_sc as plsc

N = 10000
K = 32
C = 128
NP = 10240
NW = 32
PPW = NP // NW
CPC = 4
NCH = PPW // CPC
TN = 2048
KS3 = 125


def _sc_gather_sum(feaT, idx):
    mesh = plsc.VectorSubcoreMesh(
        core_axis_name="c", subcore_axis_name="s", num_cores=2, num_subcores=16
    )

    @functools.partial(
        pl.kernel,
        out_type=jax.ShapeDtypeStruct((NP, C), jnp.float32),
        mesh=mesh,
        scratch_types=[
            pltpu.VMEM((16, 128), jnp.int32),
            pltpu.VMEM((128, C), jnp.float32),
            pltpu.VMEM((128, C), jnp.float32),
            pltpu.VMEM((CPC, C), jnp.float32),
            pltpu.VMEM((CPC, C), jnp.float32),
            pltpu.VMEM_SHARED((NP, C), jnp.float32),
            pltpu.SemaphoreType.DMA,
            pltpu.SemaphoreType.DMA,
            pltpu.SemaphoreType.DMA,
            pltpu.SemaphoreType.DMA,
        ],
    )
    def sc_kernel(
        feaT_hbm, idx_hbm, out_hbm,
        idx_v, rows0, rows1, st0, st1, tab_sh,
        gsem0, gsem1, osem0, osem1,
    ):
        sid = lax.axis_index("s")
        wid = sid * 2 + lax.axis_index("c")
        obase = wid * PPW
        tb = NP // 16
        for piece in range(tb // 128):
            off = sid * tb + piece * 128
            pltpu.sync_copy(feaT_hbm.at[pl.ds(off, 128)], rows0)
            pltpu.sync_copy(rows0, tab_sh.at[pl.ds(off, 128)])
        pltpu.sync_copy(idx_hbm.at[wid, 0], idx_v)
        plsc.subcore_barrier()

        rows = (rows0, rows1)
        stages = (st0, st1)
        gsems = (gsem0, gsem1)
        osems = (osem0, osem1)

        pltpu.async_copy(tab_sh.at[idx_v.at[0]], rows0, gsem0)
        pltpu.async_copy(tab_sh.at[idx_v.at[1]], rows1, gsem1)

        def do_chunk(c, rb, sg, gsem, osem, first):
            pltpu.make_async_copy(tab_sh.at[idx_v.at[c % 16]], rb, gsem).wait()
            if not first:
                pltpu.make_async_copy(
                    sg, out_hbm.at[pl.ds(obase + (c - 2) * CPC, CPC)], osem
                ).wait()
            for p in range(CPC):
                r0 = p * K

                def jbody(j, accs, _r0=r0):
                    return tuple(
                        accs[v] + rb[_r0 + j, pl.ds(v * 16, 16)]
                        for v in range(C // 16)
                    )

                accs = tuple(
                    rb[r0, pl.ds(v * 16, 16)] for v in range(C // 16)
                )
                accs = lax.fori_loop(1, K, jbody, accs)
                for v in range(C // 16):
                    sg[p, pl.ds(v * 16, 16)] = accs[v]
            pltpu.async_copy(
                sg, out_hbm.at[pl.ds(obase + c * CPC, CPC)], osem
            )

            @pl.when((c % 16 == 13) & (c < NCH - 3))
            def _():
                pltpu.sync_copy(idx_hbm.at[wid, c // 16 + 1], idx_v)

            @pl.when(c + 2 < NCH)
            def _():
                pltpu.async_copy(tab_sh.at[idx_v.at[(c + 2) % 16]], rb, gsem)

        for b in range(2):
            do_chunk(b, rows[b], stages[b], gsems[b], osems[b], True)

        def two_chunks(i, carry):
            c0 = i * 2
            for b in range(2):
                do_chunk(c0 + b, rows[b], stages[b], gsems[b], osems[b], False)
            return carry

        lax.fori_loop(1, NCH // 2, two_chunks, 0)
        for b in range(2):
            c = NCH - 2 + b
            pltpu.make_async_copy(
                stages[b], out_hbm.at[pl.ds(obase + c * CPC, CPC)], osems[b]
            ).wait()

    return sc_kernel(feaT, idx)


def _tc_body(s_ref, fea_ref, rel_ref, smp_ref, w1f_ref, w1x_ref, b1_ref, dw_ref, out_ref):
    w1f = w1f_ref[:]
    z = lax.dot_general(
        w1f, s_ref[:], (((1,), (1,)), ((), ())), preferred_element_type=jnp.float32
    )
    z = z - 30.0 * jnp.dot(w1f, fea_ref[:], preferred_element_type=jnp.float32)
    rel_s = jnp.sum(rel_ref[:], axis=1)
    z = z + jnp.dot(w1x_ref[:], rel_s, preferred_element_type=jnp.float32)
    z = z + 32.0 * b1_ref[:]

    smp = jnp.clip(smp_ref[:], -0.99999, 0.99999) * (5.0 / 2.0)
    coord = smp.astype(jnp.int32) + 2
    pos = coord[2:3, :] * 25 + coord[1:2, :] * 5 + coord[0:1, :]
    onehot = (
        lax.broadcasted_iota(jnp.int32, (KS3, TN), 0) == pos
    ).astype(jnp.float32)
    kern = jnp.dot(dw_ref[:], onehot, preferred_element_type=jnp.float32)
    out_ref[:] = kern * z


def _tc_combine(S, fea2d, rel3, smpT, W1f, W1x, b1c, dw):
    return pl.pallas_call(
        _tc_body,
        grid=((N + TN - 1) // TN,),
        in_specs=[
            pl.BlockSpec((TN, C), lambda i: (i, 0)),
            pl.BlockSpec((C, TN), lambda i: (0, i)),
            pl.BlockSpec((3, K, TN), lambda i: (0, 0, i)),
            pl.BlockSpec((3, TN), lambda i: (0, i)),
            pl.BlockSpec((C, C), lambda i: (0, 0)),
            pl.BlockSpec((C, 3), lambda i: (0, 0)),
            pl.BlockSpec((C, 1), lambda i: (0, 0)),
            pl.BlockSpec((C, KS3), lambda i: (0, 0)),
        ],
        out_specs=pl.BlockSpec((C, TN), lambda i: (0, i)),
        out_shape=jax.ShapeDtypeStruct((C, N), jnp.float32),
    )(S, fea2d, rel3, smpT, W1f, W1x, b1c, dw)


def kernel(sample_xyz, rel_xyz, fea, knn_idx, W1, b1, conv_dw):
    fea2d = fea[0]
    feaT = jnp.pad(fea2d.T, ((0, NP - N), (0, 0)))
    idx = knn_idx[0, :, 1:]
    idx = jnp.concatenate(
        [idx, jnp.full((N, 1), N, dtype=jnp.int32)], axis=1
    )
    idx = jnp.pad(idx, ((0, NP - N), (0, 0)), constant_values=N)
    idx = idx.reshape(NW, NCH // 16, 16, 128)

    S = _sc_gather_sum(feaT, idx)

    out2d = _tc_combine(
        S,
        fea2d,
        rel_xyz[0],
        sample_xyz[0].T,
        W1[:, :C],
        W1[:, C:],
        b1.reshape(C, 1),
        conv_dw[0].reshape(C, KS3),
    )
    return out2d[None]


# speedup vs baseline: 7.8885x; 1.0179x over previous
import functools

import jax
import jax.numpy as jnp
from jax import lax
from jax.experimental import pallas as pl
from jax.experimental.pallas import tpu as pltpu
from jax.experimental.pallas import tpu_sc as plsc

N = 10000
K = 32
C = 128
NP = 10240
NW = 32
PPW = NP // NW
CPC = 4
NCH = PPW // CPC
TN = 2048
KS3 = 125


def _sc_gather_sum(feaT, idx):
    mesh = plsc.VectorSubcoreMesh(
        core_axis_name="c", subcore_axis_name="s", num_cores=2, num_subcores=16
    )

    @functools.partial(
        pl.kernel,
        out_type=jax.ShapeDtypeStruct((NP, C), jnp.float32),
        mesh=mesh,
        scratch_types=[
            pltpu.VMEM((NCH, 128), jnp.int32),
            pltpu.VMEM((128, C), jnp.float32),
            pltpu.VMEM((128, C), jnp.float32),
            pltpu.VMEM((CPC, C), jnp.float32),
            pltpu.VMEM((CPC, C), jnp.float32),
            pltpu.VMEM_SHARED((NP, C), jnp.float32),
            pltpu.SemaphoreType.DMA,
            pltpu.SemaphoreType.DMA,
            pltpu.SemaphoreType.DMA,
            pltpu.SemaphoreType.DMA,
        ],
    )
    def sc_kernel(
        feaT_hbm, idx_hbm, out_hbm,
        idx_v, rows0, rows1, st0, st1, tab_sh,
        gsem0, gsem1, osem0, osem1,
    ):
        sid = lax.axis_index("s")
        wid = sid * 2 + lax.axis_index("c")
        obase = wid * PPW
        tb = NP // 16
        for piece in range(tb // 128):
            off = sid * tb + piece * 128
            pltpu.sync_copy(feaT_hbm.at[pl.ds(off, 128)], rows0)
            pltpu.sync_copy(rows0, tab_sh.at[pl.ds(off, 128)])
        pltpu.sync_copy(idx_hbm.at[wid], idx_v)
        plsc.subcore_barrier()

        rows = (rows0, rows1)
        stages = (st0, st1)
        gsems = (gsem0, gsem1)
        osems = (osem0, osem1)

        pltpu.async_copy(tab_sh.at[idx_v.at[0]], rows0, gsem0)
        pltpu.async_copy(tab_sh.at[idx_v.at[1]], rows1, gsem1)

        def do_chunk(c, rb, sg, gsem, osem, first):
            pltpu.make_async_copy(tab_sh.at[idx_v.at[c]], rb, gsem).wait()
            if not first:
                pltpu.make_async_copy(
                    sg, out_hbm.at[pl.ds(obase + (c - 2) * CPC, CPC)], osem
                ).wait()
            for p in range(CPC):
                r0 = p * K

                def jbody(j, accs, _r0=r0):
                    return tuple(
                        accs[v] + rb[_r0 + j, pl.ds(v * 16, 16)]
                        for v in range(C // 16)
                    )

                accs = tuple(
                    rb[r0, pl.ds(v * 16, 16)] for v in range(C // 16)
                )
                accs = lax.fori_loop(1, K, jbody, accs)
                for v in range(C // 16):
                    sg[p, pl.ds(v * 16, 16)] = accs[v]
            pltpu.async_copy(
                sg, out_hbm.at[pl.ds(obase + c * CPC, CPC)], osem
            )

            @pl.when(c + 2 < NCH)
            def _():
                pltpu.async_copy(tab_sh.at[idx_v.at[c + 2]], rb, gsem)

        for b in range(2):
            do_chunk(b, rows[b], stages[b], gsems[b], osems[b], True)

        def two_chunks(i, carry):
            c0 = i * 2
            for b in range(2):
                do_chunk(c0 + b, rows[b], stages[b], gsems[b], osems[b], False)
            return carry

        lax.fori_loop(1, NCH // 2, two_chunks, 0)
        for b in range(2):
            c = NCH - 2 + b
            pltpu.make_async_copy(
                stages[b], out_hbm.at[pl.ds(obase + c * CPC, CPC)], osems[b]
            ).wait()

    return sc_kernel(feaT, idx)


def _tc_body(s_ref, fea_ref, rel_ref, smp_ref, w1f_ref, w1x_ref, b1_ref, dw_ref, out_ref):
    w1f = w1f_ref[:]
    z = lax.dot_general(
        w1f, s_ref[:], (((1,), (1,)), ((), ())), preferred_element_type=jnp.float32
    )
    z = z - 30.0 * jnp.dot(w1f, fea_ref[:], preferred_element_type=jnp.float32)
    rel_s = jnp.sum(rel_ref[:], axis=1)
    z = z + jnp.dot(w1x_ref[:], rel_s, preferred_element_type=jnp.float32)
    z = z + 32.0 * b1_ref[:]

    smp = jnp.clip(smp_ref[:], -0.99999, 0.99999) * (5.0 / 2.0)
    coord = smp.astype(jnp.int32) + 2
    pos = coord[2:3, :] * 25 + coord[1:2, :] * 5 + coord[0:1, :]
    onehot = (
        lax.broadcasted_iota(jnp.int32, (KS3, TN), 0) == pos
    ).astype(jnp.float32)
    kern = jnp.dot(dw_ref[:], onehot, preferred_element_type=jnp.float32)
    out_ref[:] = kern * z


def _tc_combine(S, fea2d, rel3, smpT, W1f, W1x, b1c, dw):
    return pl.pallas_call(
        _tc_body,
        grid=((N + TN - 1) // TN,),
        in_specs=[
            pl.BlockSpec((TN, C), lambda i: (i, 0)),
            pl.BlockSpec((C, TN), lambda i: (0, i)),
            pl.BlockSpec((3, K, TN), lambda i: (0, 0, i)),
            pl.BlockSpec((3, TN), lambda i: (0, i)),
            pl.BlockSpec((C, C), lambda i: (0, 0)),
            pl.BlockSpec((C, 3), lambda i: (0, 0)),
            pl.BlockSpec((C, 1), lambda i: (0, 0)),
            pl.BlockSpec((C, KS3), lambda i: (0, 0)),
        ],
        out_specs=pl.BlockSpec((C, TN), lambda i: (0, i)),
        out_shape=jax.ShapeDtypeStruct((C, N), jnp.float32),
    )(S, fea2d, rel3, smpT, W1f, W1x, b1c, dw)


def kernel(sample_xyz, rel_xyz, fea, knn_idx, W1, b1, conv_dw):
    fea2d = fea[0]
    feaT = jnp.pad(fea2d.T, ((0, NP - N), (0, 0)))
    idx = knn_idx[0, :, 1:]
    idx = jnp.concatenate(
        [idx, jnp.full((N, 1), N, dtype=jnp.int32)], axis=1
    )
    idx = jnp.pad(idx, ((0, NP - N), (0, 0)), constant_values=N)
    idx = idx.reshape(NW, NCH, 128)

    S = _sc_gather_sum(feaT, idx)

    out2d = _tc_combine(
        S,
        fea2d,
        rel_xyz[0],
        sample_xyz[0].T,
        W1[:, :C],
        W1[:, C:],
        b1.reshape(C, 1),
        conv_dw[0].reshape(C, KS3),
    )
    return out2d[None]
